# baseline (speedup 1.0000x reference)
import functools

import jax
import jax.numpy as jnp
from jax import lax
from jax.experimental import pallas as pl
from jax.experimental.pallas import tpu as pltpu
from jax.experimental.pallas import tpu_sc as plsc

WIN = 12
N = 10000
NE = 320000
C = 128
KER = 3

NSUB = 16
NCORE = 2
NEPAD = 327680
NPAD = 10240
ECH = 2048
EA = NEPAD // NSUB
EC = NEPAD // (NSUB * NCORE)
RPT = NPAD // NSUB
TPC = WIN // NCORE

_MESH = plsc.VectorSubcoreMesh(core_axis_name="c", subcore_axis_name="s")
_SC_PARAMS = pltpu.CompilerParams(needs_layout_passes=False)


def _rsqrt_newton(x):
    i = plsc.bitcast(x, jnp.int32)
    y = plsc.bitcast(jnp.int32(0x5F3759DF) - (i >> 1), jnp.float32)
    for _ in range(3):
        y = y * (1.5 - 0.5 * x * y * y)
    return y


@functools.partial(
    pl.kernel,
    out_type=jax.ShapeDtypeStruct((NEPAD,), jnp.float32),
    mesh=_MESH,
    compiler_params=_SC_PARAMS,
    scratch_types=[
        pltpu.VMEM((NPAD,), jnp.float32),
        pltpu.VMEM((640,), jnp.float32),
        pltpu.VMEM((NPAD,), jnp.float32),
        pltpu.VMEM((ECH,), jnp.int32),
        pltpu.VMEM((ECH,), jnp.int32),
        pltpu.VMEM((ECH,), jnp.float32),
        pltpu.VMEM((ECH,), jnp.float32),
        pltpu.VMEM_SHARED((NSUB * NPAD,), jnp.float32),
        pltpu.VMEM_SHARED((NPAD,), jnp.float32),
    ],
)
def _norm_kernel(src_hbm, dst_hbm, e_hbm, norm_hbm,
                 degpart, d640, disv, ibuf, jbuf, wbuf, obuf, shacc, shdis):
    s = lax.axis_index("s")
    c = lax.axis_index("c")
    wid = c * NSUB + s

    zero16 = jnp.zeros((16,), jnp.float32)

    def zloop(i, _):
        degpart[pl.ds(pl.multiple_of(i * 16, 16), 16)] = zero16
        return _
    lax.fori_loop(0, NPAD // 16, zloop, None)

    abase = s * EA

    def achunk(k, _):
        off = pl.multiple_of(abase + k * ECH, 128)
        pltpu.sync_copy(dst_hbm.at[pl.ds(off, ECH)], jbuf)
        pltpu.sync_copy(e_hbm.at[pl.ds(off, ECH)], wbuf)

        def ae(i, _):
            b = pl.multiple_of(i * 16, 16)
            d16 = jbuf[pl.ds(b, 16)]
            w16 = jnp.maximum(wbuf[pl.ds(b, 16)], 0.0)
            plsc.addupdate_scatter(degpart, [d16], w16)
            return _
        lax.fori_loop(0, ECH // 16, ae, None)
        return _
    lax.fori_loop(0, EA // ECH, achunk, None)

    pltpu.sync_copy(degpart, shacc.at[pl.ds(pl.multiple_of(s * NPAD, 128), NPAD)])
    plsc.subcore_barrier()
    for r in range(NSUB):
        pltpu.sync_copy(shacc.at[pl.ds(pl.multiple_of(r * NPAD + s * 640, 128), 640)],
                        degpart.at[pl.ds(r * 640, 640)])

    def bsum(v, _):
        b = pl.multiple_of(v * 16, 16)
        acc = degpart[pl.ds(b, 16)]
        for r in range(1, NSUB):
            acc = acc + degpart[pl.ds(pl.multiple_of(r * 640 + v * 16, 16), 16)]
        y = _rsqrt_newton(acc)
        d640[pl.ds(b, 16)] = jnp.where(acc > 0.0, y, 0.0)
        return _
    lax.fori_loop(0, 640 // 16, bsum, None)
    pltpu.sync_copy(d640, shdis.at[pl.ds(pl.multiple_of(s * 640, 128), 640)])
    plsc.subcore_barrier()
    pltpu.sync_copy(shdis, disv)

    cbase = wid * EC

    def cchunk(k, _):
        off = pl.multiple_of(cbase + k * ECH, 128)
        pltpu.sync_copy(src_hbm.at[pl.ds(off, ECH)], ibuf)
        pltpu.sync_copy(dst_hbm.at[pl.ds(off, ECH)], jbuf)
        pltpu.sync_copy(e_hbm.at[pl.ds(off, ECH)], wbuf)

        def ce(i, _):
            b = pl.multiple_of(i * 16, 16)
            s16 = ibuf[pl.ds(b, 16)]
            d16 = jbuf[pl.ds(b, 16)]
            w16 = jnp.maximum(wbuf[pl.ds(b, 16)], 0.0)
            dis_s = plsc.load_gather(disv, [s16])
            dis_d = plsc.load_gather(disv, [d16])
            obuf[pl.ds(b, 16)] = -(dis_s * w16 * dis_d)
            return _
        lax.fori_loop(0, ECH // 16, ce, None)
        pltpu.sync_copy(obuf, norm_hbm.at[pl.ds(off, ECH)])
        return _
    lax.fori_loop(0, EC // ECH, cchunk, None)


@functools.partial(
    pl.kernel,
    out_type=jax.ShapeDtypeStruct((WIN * N, C), jnp.float32),
    mesh=_MESH,
    compiler_params=_SC_PARAMS,
    scratch_types=[
        pltpu.VMEM((ECH,), jnp.int32),
        pltpu.VMEM((ECH // 128, 128), jnp.int32),
        pltpu.VMEM((ECH,), jnp.float32),
        pltpu.VMEM((128, C), jnp.float32),
        pltpu.VMEM((128, C), jnp.float32),
        pltpu.VMEM_SHARED((NPAD, C), jnp.float32),
        pltpu.SemaphoreType.DMA,
        pltpu.SemaphoreType.DMA,
    ],
)
def _spmm_kernel(xq_hbm, src_hbm, dst2_hbm, norm_hbm, zeros_hbm, tx_hbm,
                 sbuf, dbuf, nbuf, gbufa, gbufb, acc, gsa, gsb):
    s = lax.axis_index("s")
    c = lax.axis_index("c")
    abase = s * EA

    rbase = pl.multiple_of(s * RPT, 8)
    for j in range(TPC):
        t = c * TPC + j
        for z in range(RPT // 128):
            pltpu.sync_copy(zeros_hbm.at[pl.ds(0, 128)],
                            acc.at[pl.ds(rbase + z * 128, 128)])
        plsc.subcore_barrier()

        def superchunk(sc_i, _):
            off = pl.multiple_of(abase + sc_i * ECH, 128)
            roff = pl.multiple_of(s * (EA // 128) + sc_i * (ECH // 128), 8)
            pltpu.sync_copy(src_hbm.at[pl.ds(off, ECH)], sbuf)
            pltpu.sync_copy(dst2_hbm.at[pl.ds(roff, ECH // 128)], dbuf)
            pltpu.sync_copy(norm_hbm.at[pl.ds(off, ECH)], nbuf)

            def bias(i, _):
                b = pl.multiple_of(i * 16, 16)
                sbuf[pl.ds(b, 16)] = sbuf[pl.ds(b, 16)] + t * N
                return _
            lax.fori_loop(0, ECH // 16, bias, None)

            def gather_fire(k, buf, sem):
                idx = sbuf.at[pl.ds(k * 128, 128)]
                pltpu.async_copy(xq_hbm.at[idx], buf, sem)

            def gather_wait(buf, sem):
                pltpu.make_async_copy(xq_hbm.at[pl.ds(0, 128)], buf, sem).wait()

            def scale(gb, k):
                def egroup(g, _):
                    nv16 = nbuf[pl.ds(pl.multiple_of(k * 128 + g * 16, 16), 16)]
                    for e in range(16):
                        row = g * 16 + e
                        nv = nv16[e]
                        for jj in range(C // 16):
                            sl = pl.ds(jj * 16, 16)
                            gb[row, sl] = gb[row, sl] * nv
                    return _
                lax.fori_loop(0, 8, egroup, None)

            gather_fire(0, gbufa, gsa)
            gather_fire(1, gbufb, gsb)

            npair = ECH // 256
            def pair(kk, _):
                k0 = kk * 2
                gather_wait(gbufa, gsa)
                scale(gbufa, k0)
                pltpu.sync_copy(gbufa, acc.at[dbuf.at[k0]], add=True)

                @pl.when(kk < npair - 1)
                def _fa():
                    gather_fire(k0 + 2, gbufa, gsa)

                gather_wait(gbufb, gsb)
                scale(gbufb, k0 + 1)
                pltpu.sync_copy(gbufb, acc.at[dbuf.at[k0 + 1]], add=True)

                @pl.when(kk < npair - 1)
                def _fb():
                    gather_fire(k0 + 3, gbufb, gsb)
                return _
            lax.fori_loop(0, npair, pair, None)
            return _
        lax.fori_loop(0, EA // ECH, superchunk, None)
        plsc.subcore_barrier()

        obase = pl.multiple_of(t * N + rbase, 8)

        @pl.when(s < NSUB - 1)
        def _copy_full():
            pltpu.sync_copy(acc.at[pl.ds(rbase, RPT)],
                            tx_hbm.at[pl.ds(obase, RPT)])

        @pl.when(s == NSUB - 1)
        def _copy_tail():
            last = N - (NSUB - 1) * RPT
            pltpu.sync_copy(acc.at[pl.ds(rbase, last)],
                            tx_hbm.at[pl.ds(obase, last)])

        if j < TPC - 1:
            def adv(i, _):
                b = pl.multiple_of(i * 16, 16)
                sbuf[pl.ds(b, 16)] = sbuf[pl.ds(b, 16)] + N
                return _
            lax.fori_loop(0, EA // 16, adv, None)


BN = 400


def _tc_body(x_ref, tx_ref, w_ref, b_ref, cw_ref, cb_ref, out_ref):
    hs = []
    for t in range(WIN):
        h = jnp.dot(x_ref[t], w_ref[t, 0], preferred_element_type=jnp.float32)
        h = h + jnp.dot(tx_ref[t], w_ref[t, 1], preferred_element_type=jnp.float32)
        h = h + b_ref[t][None, :]
        hs.append(h)
    for t in range(WIN):
        o = None
        for kk in range(KER):
            tt = t - 1 + kk
            if 0 <= tt < WIN:
                term = jnp.dot(hs[tt], cw_ref[kk], preferred_element_type=jnp.float32)
                o = term if o is None else o + term
        o = o + cb_ref[0][None, :]
        o = jnp.where(o >= 0.0, o, 0.01 * o)
        out_ref[:, t, :] = o


def kernel(x_list, A, E, batch_size, gcn_W, gcn_b, conv_w, conv_b):
    del batch_size
    pad = NEPAD - NE
    src = jnp.concatenate([A[0].astype(jnp.int32), jnp.zeros((pad,), jnp.int32)])
    dst = jnp.concatenate([A[1].astype(jnp.int32), jnp.zeros((pad,), jnp.int32)])
    ew = jnp.concatenate([E.astype(jnp.float32), jnp.zeros((pad,), jnp.float32)])

    norm = _norm_kernel(src, dst, ew)

    x2d = x_list.reshape(WIN * N, C)
    xq = x2d
    zeros = jnp.zeros((128, C), jnp.float32)
    tx2d = _spmm_kernel(xq, src, dst.reshape(NEPAD // 128, 128), norm, zeros)

    cwT = jnp.transpose(conv_w, (2, 1, 0))
    cb2 = conv_b.reshape(1, C)
    x3 = x_list
    tx3 = tx2d.reshape(WIN, N, C)

    out = pl.pallas_call(
        _tc_body,
        grid=(N // BN,),
        in_specs=[
            pl.BlockSpec((WIN, BN, C), lambda i: (0, i, 0)),
            pl.BlockSpec((WIN, BN, C), lambda i: (0, i, 0)),
            pl.BlockSpec((WIN, 2, C, C), lambda i: (0, 0, 0, 0)),
            pl.BlockSpec((WIN, C), lambda i: (0, 0)),
            pl.BlockSpec((KER, C, C), lambda i: (0, 0, 0)),
            pl.BlockSpec((1, C), lambda i: (0, 0)),
        ],
        out_specs=pl.BlockSpec((BN, WIN, C), lambda i: (i, 0, 0)),
        out_shape=jax.ShapeDtypeStruct((N, WIN, C), jnp.float32),
    )(x3, tx3, gcn_W, gcn_b, cwT, cb2)
    return out


# speedup vs baseline: 1.0259x; 1.0259x over previous
import functools

import jax
import jax.numpy as jnp
from jax import lax
from jax.experimental import pallas as pl
from jax.experimental.pallas import tpu as pltpu
from jax.experimental.pallas import tpu_sc as plsc

WIN = 12
N = 10000
NE = 320000
C = 128
KER = 3

NSUB = 16
NCORE = 2
NEPAD = 327680
NPAD = 10240
ECH = 2048
SECH = 4096
EA = NEPAD // NSUB
EC = NEPAD // (NSUB * NCORE)
RPT = NPAD // NSUB
TPC = WIN // NCORE

_MESH = plsc.VectorSubcoreMesh(core_axis_name="c", subcore_axis_name="s")
_SC_PARAMS = pltpu.CompilerParams(needs_layout_passes=False)


def _rsqrt_newton(x):
    i = plsc.bitcast(x, jnp.int32)
    y = plsc.bitcast(jnp.int32(0x5F3759DF) - (i >> 1), jnp.float32)
    for _ in range(3):
        y = y * (1.5 - 0.5 * x * y * y)
    return y


@functools.partial(
    pl.kernel,
    out_type=jax.ShapeDtypeStruct((NEPAD,), jnp.float32),
    mesh=_MESH,
    compiler_params=_SC_PARAMS,
    scratch_types=[
        pltpu.VMEM((NPAD,), jnp.float32),
        pltpu.VMEM((640,), jnp.float32),
        pltpu.VMEM((NPAD,), jnp.float32),
        pltpu.VMEM((ECH,), jnp.int32),
        pltpu.VMEM((ECH,), jnp.int32),
        pltpu.VMEM((ECH,), jnp.float32),
        pltpu.VMEM((ECH,), jnp.float32),
        pltpu.VMEM_SHARED((NSUB * NPAD,), jnp.float32),
        pltpu.VMEM_SHARED((NPAD,), jnp.float32),
    ],
)
def _norm_kernel(src_hbm, dst_hbm, e_hbm, norm_hbm,
                 degpart, d640, disv, ibuf, jbuf, wbuf, obuf, shacc, shdis):
    s = lax.axis_index("s")
    c = lax.axis_index("c")
    wid = c * NSUB + s

    zero16 = jnp.zeros((16,), jnp.float32)

    def zloop(i, _):
        degpart[pl.ds(pl.multiple_of(i * 16, 16), 16)] = zero16
        return _
    lax.fori_loop(0, NPAD // 16, zloop, None)

    abase = s * EA

    def achunk(k, _):
        off = pl.multiple_of(abase + k * ECH, 128)
        pltpu.sync_copy(dst_hbm.at[pl.ds(off, ECH)], jbuf)
        pltpu.sync_copy(e_hbm.at[pl.ds(off, ECH)], wbuf)

        def ae(i, _):
            b = pl.multiple_of(i * 16, 16)
            d16 = jbuf[pl.ds(b, 16)]
            w16 = jnp.maximum(wbuf[pl.ds(b, 16)], 0.0)
            plsc.addupdate_scatter(degpart, [d16], w16)
            return _
        lax.fori_loop(0, ECH // 16, ae, None)
        return _
    lax.fori_loop(0, EA // ECH, achunk, None)

    pltpu.sync_copy(degpart, shacc.at[pl.ds(pl.multiple_of(s * NPAD, 128), NPAD)])
    plsc.subcore_barrier()
    for r in range(NSUB):
        pltpu.sync_copy(shacc.at[pl.ds(pl.multiple_of(r * NPAD + s * 640, 128), 640)],
                        degpart.at[pl.ds(r * 640, 640)])

    def bsum(v, _):
        b = pl.multiple_of(v * 16, 16)
        acc = degpart[pl.ds(b, 16)]
        for r in range(1, NSUB):
            acc = acc + degpart[pl.ds(pl.multiple_of(r * 640 + v * 16, 16), 16)]
        y = _rsqrt_newton(acc)
        d640[pl.ds(b, 16)] = jnp.where(acc > 0.0, y, 0.0)
        return _
    lax.fori_loop(0, 640 // 16, bsum, None)
    pltpu.sync_copy(d640, shdis.at[pl.ds(pl.multiple_of(s * 640, 128), 640)])
    plsc.subcore_barrier()
    pltpu.sync_copy(shdis, disv)

    cbase = wid * EC

    def cchunk(k, _):
        off = pl.multiple_of(cbase + k * ECH, 128)
        pltpu.sync_copy(src_hbm.at[pl.ds(off, ECH)], ibuf)
        pltpu.sync_copy(dst_hbm.at[pl.ds(off, ECH)], jbuf)
        pltpu.sync_copy(e_hbm.at[pl.ds(off, ECH)], wbuf)

        def ce(i, _):
            b = pl.multiple_of(i * 16, 16)
            s16 = ibuf[pl.ds(b, 16)]
            d16 = jbuf[pl.ds(b, 16)]
            w16 = jnp.maximum(wbuf[pl.ds(b, 16)], 0.0)
            dis_s = plsc.load_gather(disv, [s16])
            dis_d = plsc.load_gather(disv, [d16])
            obuf[pl.ds(b, 16)] = -(dis_s * w16 * dis_d)
            return _
        lax.fori_loop(0, ECH // 16, ce, None)
        pltpu.sync_copy(obuf, norm_hbm.at[pl.ds(off, ECH)])
        return _
    lax.fori_loop(0, EC // ECH, cchunk, None)


@functools.partial(
    pl.kernel,
    out_type=jax.ShapeDtypeStruct((WIN * N, C), jnp.float32),
    mesh=_MESH,
    compiler_params=_SC_PARAMS,
    scratch_types=[
        pltpu.VMEM((SECH,), jnp.int32),
        pltpu.VMEM((SECH // 128, 128), jnp.int32),
        pltpu.VMEM((SECH,), jnp.float32),
        pltpu.VMEM((128, C), jnp.float32),
        pltpu.VMEM((128, C), jnp.float32),
        pltpu.VMEM_SHARED((NPAD, C), jnp.float32),
        pltpu.SemaphoreType.DMA,
        pltpu.SemaphoreType.DMA,
    ],
)
def _spmm_kernel(xq_hbm, src_hbm, dst2_hbm, norm_hbm, zeros_hbm, tx_hbm,
                 sbuf, dbuf, nbuf, gbufa, gbufb, acc, gsa, gsb):
    s = lax.axis_index("s")
    c = lax.axis_index("c")
    abase = s * EA

    rbase = pl.multiple_of(s * RPT, 8)
    for j in range(TPC):
        t = c * TPC + j
        for z in range(RPT // 128):
            pltpu.sync_copy(zeros_hbm.at[pl.ds(0, 128)],
                            acc.at[pl.ds(rbase + z * 128, 128)])
        plsc.subcore_barrier()

        def superchunk(sc_i, _):
            off = pl.multiple_of(abase + sc_i * SECH, 128)
            roff = pl.multiple_of(s * (EA // 128) + sc_i * (SECH // 128), 8)
            pltpu.sync_copy(src_hbm.at[pl.ds(off, SECH)], sbuf)
            pltpu.sync_copy(dst2_hbm.at[pl.ds(roff, SECH // 128)], dbuf)
            pltpu.sync_copy(norm_hbm.at[pl.ds(off, SECH)], nbuf)

            def bias(i, _):
                b = pl.multiple_of(i * 16, 16)
                sbuf[pl.ds(b, 16)] = sbuf[pl.ds(b, 16)] + t * N
                return _
            lax.fori_loop(0, SECH // 16, bias, None)

            def gather_fire(k, buf, sem):
                idx = sbuf.at[pl.ds(k * 128, 128)]
                pltpu.async_copy(xq_hbm.at[idx], buf, sem)

            def gather_wait(buf, sem):
                pltpu.make_async_copy(xq_hbm.at[pl.ds(0, 128)], buf, sem).wait()

            def scale(gb, k):
                def egroup(g, _):
                    nv16 = nbuf[pl.ds(pl.multiple_of(k * 128 + g * 16, 16), 16)]
                    for e in range(16):
                        row = g * 16 + e
                        nv = nv16[e]
                        for jj in range(C // 16):
                            sl = pl.ds(jj * 16, 16)
                            gb[row, sl] = gb[row, sl] * nv
                    return _
                lax.fori_loop(0, 8, egroup, None)

            gather_fire(0, gbufa, gsa)
            gather_fire(1, gbufb, gsb)

            npair = SECH // 256
            def pair(kk, _):
                k0 = kk * 2
                gather_wait(gbufa, gsa)
                scale(gbufa, k0)
                pltpu.sync_copy(gbufa, acc.at[dbuf.at[k0]], add=True)

                @pl.when(kk < npair - 1)
                def _fa():
                    gather_fire(k0 + 2, gbufa, gsa)

                gather_wait(gbufb, gsb)
                scale(gbufb, k0 + 1)
                pltpu.sync_copy(gbufb, acc.at[dbuf.at[k0 + 1]], add=True)

                @pl.when(kk < npair - 1)
                def _fb():
                    gather_fire(k0 + 3, gbufb, gsb)
                return _
            lax.fori_loop(0, npair, pair, None)
            return _
        lax.fori_loop(0, EA // SECH, superchunk, None)
        plsc.subcore_barrier()

        obase = pl.multiple_of(t * N + rbase, 8)

        @pl.when(s < NSUB - 1)
        def _copy_full():
            pltpu.sync_copy(acc.at[pl.ds(rbase, RPT)],
                            tx_hbm.at[pl.ds(obase, RPT)])

        @pl.when(s == NSUB - 1)
        def _copy_tail():
            last = N - (NSUB - 1) * RPT
            pltpu.sync_copy(acc.at[pl.ds(rbase, last)],
                            tx_hbm.at[pl.ds(obase, last)])

        if j < TPC - 1:
            def adv(i, _):
                b = pl.multiple_of(i * 16, 16)
                sbuf[pl.ds(b, 16)] = sbuf[pl.ds(b, 16)] + N
                return _
            lax.fori_loop(0, EA // 16, adv, None)


BN = 400


def _tc_body(x_ref, tx_ref, w_ref, b_ref, cw_ref, cb_ref, out_ref):
    hs = []
    for t in range(WIN):
        h = jnp.dot(x_ref[t], w_ref[t, 0], preferred_element_type=jnp.float32)
        h = h + jnp.dot(tx_ref[t], w_ref[t, 1], preferred_element_type=jnp.float32)
        h = h + b_ref[t][None, :]
        hs.append(h)
    for t in range(WIN):
        o = None
        for kk in range(KER):
            tt = t - 1 + kk
            if 0 <= tt < WIN:
                term = jnp.dot(hs[tt], cw_ref[kk], preferred_element_type=jnp.float32)
                o = term if o is None else o + term
        o = o + cb_ref[0][None, :]
        o = jnp.where(o >= 0.0, o, 0.01 * o)
        out_ref[:, t, :] = o


def kernel(x_list, A, E, batch_size, gcn_W, gcn_b, conv_w, conv_b):
    del batch_size
    pad = NEPAD - NE
    src = jnp.concatenate([A[0].astype(jnp.int32), jnp.zeros((pad,), jnp.int32)])
    dst = jnp.concatenate([A[1].astype(jnp.int32), jnp.zeros((pad,), jnp.int32)])
    ew = jnp.concatenate([E.astype(jnp.float32), jnp.zeros((pad,), jnp.float32)])

    norm = _norm_kernel(src, dst, ew)

    x2d = x_list.reshape(WIN * N, C)
    xq = x2d
    zeros = jnp.zeros((128, C), jnp.float32)
    tx2d = _spmm_kernel(xq, src, dst.reshape(NEPAD // 128, 128), norm, zeros)

    cwT = jnp.transpose(conv_w, (2, 1, 0))
    cb2 = conv_b.reshape(1, C)
    x3 = x_list
    tx3 = tx2d.reshape(WIN, N, C)

    out = pl.pallas_call(
        _tc_body,
        grid=(N // BN,),
        in_specs=[
            pl.BlockSpec((WIN, BN, C), lambda i: (0, i, 0)),
            pl.BlockSpec((WIN, BN, C), lambda i: (0, i, 0)),
            pl.BlockSpec((WIN, 2, C, C), lambda i: (0, 0, 0, 0)),
            pl.BlockSpec((WIN, C), lambda i: (0, 0)),
            pl.BlockSpec((KER, C, C), lambda i: (0, 0, 0)),
            pl.BlockSpec((1, C), lambda i: (0, 0)),
        ],
        out_specs=pl.BlockSpec((BN, WIN, C), lambda i: (i, 0, 0)),
        out_shape=jax.ShapeDtypeStruct((N, WIN, C), jnp.float32),
    )(x3, tx3, gcn_W, gcn_b, cwT, cb2)
    return out
